# baseline (speedup 1.0000x reference)
import functools

import jax
import jax.numpy as jnp
from jax import lax
from jax.experimental import pallas as pl
from jax.experimental.pallas import tpu as pltpu
from jax.experimental.pallas import tpu_sc as plsc

NC = 2
NS = 16
NW = NC * NS


def _fill(ref, value, n):
    def body(i, _):
        ref[pl.ds(pl.multiple_of(i * 16, 16), 16)] = jnp.full(
            (16,), value, jnp.float32)
        return 0
    lax.fori_loop(0, n // 16, body, 0)


def _fill2d(ref, value, rows, cols):
    def body(i, _):
        r = i // (cols // 16)
        q = pl.multiple_of((i % (cols // 16)) * 16, 16)
        ref[r, pl.ds(q, 16)] = jnp.full((16,), value, jnp.float32)
        return 0
    lax.fori_loop(0, rows * (cols // 16), body, 0)


def _make_sc_kernels(n_pad, ec, fh):
    nslice = n_pad // NS
    mesh = plsc.VectorSubcoreMesh(core_axis_name="c", subcore_axis_name="s")
    sc_params = pltpu.CompilerParams(
        use_tc_tiling_on_sc=False, needs_layout_passes=False)
    sds = jax.ShapeDtypeStruct

    @functools.partial(
        pl.kernel,
        out_type=(sds((n_pad,), jnp.float32), sds((n_pad,), jnp.float32)),
        mesh=mesh,
        compiler_params=sc_params,
        scratch_types=[
            pltpu.VMEM((ec, 128), jnp.int32),
            pltpu.VMEM((128,), jnp.float32),
            pltpu.VMEM((nslice,), jnp.float32),
            pltpu.VMEM_SHARED((n_pad,), jnp.float32),
            pltpu.SemaphoreType.DMA,
        ],
    )
    def deg_kernel(dst_hbm, out_a, out_b, dst_v, ones_v, buf_v, acc_sh, sem):
        c = lax.axis_index("c")
        s = lax.axis_index("s")
        gid = c * NS + s
        _fill(ones_v, 1.0, 128)
        _fill(buf_v, 0.0, nslice)
        pltpu.sync_copy(buf_v, acc_sh.at[pl.ds(s * nslice, nslice)])
        plsc.subcore_barrier()
        pltpu.sync_copy(dst_hbm.at[pl.ds(gid * ec, ec)], dst_v)

        def fire(i, _):
            pltpu.async_copy(ones_v, acc_sh.at[dst_v.at[i]], sem, add=True)
            return 0
        lax.fori_loop(0, ec, fire, 0)

        def drain(i, _):
            pltpu.make_async_copy(ones_v, acc_sh.at[dst_v.at[i]], sem).wait()
            return 0
        lax.fori_loop(0, ec, drain, 0)
        plsc.subcore_barrier()
        pltpu.sync_copy(acc_sh.at[pl.ds(s * nslice, nslice)], buf_v)

        @pl.when(c == 0)
        def _():
            pltpu.sync_copy(buf_v, out_a.at[pl.ds(s * nslice, nslice)])

        @pl.when(c == 1)
        def _():
            pltpu.sync_copy(buf_v, out_b.at[pl.ds(s * nslice, nslice)])

    nb = 5
    assert ec % nb == 0

    @functools.partial(
        pl.kernel,
        out_type=(sds((n_pad, fh), jnp.float32), sds((n_pad, fh), jnp.float32)),
        mesh=mesh,
        compiler_params=sc_params,
        scratch_types=[
            pltpu.VMEM_SHARED((n_pad, fh), jnp.float32),
            pltpu.VMEM_SHARED((n_pad, fh), jnp.float32),
        ] + [pltpu.VMEM((128, fh), jnp.float32) for _ in range(nb)]
          + [pltpu.VMEM((1, 128), jnp.int32) for _ in range(2 * nb)]
          + [pltpu.SemaphoreType.DMA for _ in range(4 * nb)],
    )
    def agg1_kernel(u_hbm, src_hbm, dst_hbm, out_a, out_b,
                    u_sh, acc_sh, *rest):
        rows = rest[:nb]
        srcr = rest[nb:2 * nb]
        dstr = rest[2 * nb:3 * nb]
        gsem = rest[3 * nb:4 * nb]
        ssem = rest[4 * nb:5 * nb]
        srcsem = rest[5 * nb:6 * nb]
        dstsem = rest[6 * nb:7 * nb]
        c = lax.axis_index("c")
        s = lax.axis_index("s")
        gid = c * NS + s

        def src_load(b, j):
            pltpu.async_copy(src_hbm.at[pl.ds(gid * ec + j, 1)], srcr[b],
                             srcsem[b])

        def src_wait(b):
            pltpu.make_async_copy(src_hbm.at[pl.ds(gid * ec, 1)], srcr[b],
                                  srcsem[b]).wait()

        def dst_load(b, j):
            pltpu.async_copy(dst_hbm.at[pl.ds(gid * ec + j, 1)], dstr[b],
                             dstsem[b])

        def dst_wait(b):
            pltpu.make_async_copy(dst_hbm.at[pl.ds(gid * ec, 1)], dstr[b],
                                  dstsem[b]).wait()

        pltpu.sync_copy(u_hbm.at[pl.ds(s * nslice, nslice)],
                        u_sh.at[pl.ds(s * nslice, nslice)])
        _fill2d(rows[0], 0.0, 128, fh)

        def zero(k, _):
            pltpu.sync_copy(rows[0], acc_sh.at[pl.ds(s * nslice + k * 128, 128)])
            return 0
        lax.fori_loop(0, nslice // 128, zero, 0)
        plsc.subcore_barrier()

        def gather(b, j):
            @pl.when(j % 3 == 0)
            def _():
                pltpu.async_copy(u_hbm.at[srcr[b].at[0]], rows[b], gsem[b])

            @pl.when(j % 3 != 0)
            def _():
                pltpu.async_copy(u_sh.at[srcr[b].at[0]], rows[b], gsem[b])

        for b in range(nb):
            src_load(b, b)
            dst_load(b, b)
        for b in range(nb):
            src_wait(b)
            gather(b, b)

        def round_body(r, _):
            for b in range(nb):
                j = r * nb + b
                pltpu.make_async_copy(
                    u_sh.at[srcr[b].at[0]], rows[b], gsem[b]).wait()

                @pl.when(j + nb < ec)
                def _(b=b, j=j):
                    src_load(b, j + nb)
                dst_wait(b)
                pltpu.async_copy(
                    rows[b], acc_sh.at[dstr[b].at[0]], ssem[b], add=True)
            for b in range(nb):
                j = r * nb + b

                @pl.when(j + nb < ec)
                def _(b=b, j=j):
                    pltpu.make_async_copy(
                        rows[b], acc_sh.at[dstr[b].at[0]], ssem[b]).wait()
                    dst_load(b, j + nb)
                    src_wait(b)
                    gather(b, j + nb)
            return 0
        lax.fori_loop(0, ec // nb, round_body, 0)
        for b in range(nb):
            pltpu.make_async_copy(
                rows[b], acc_sh.at[dstr[b].at[0]], ssem[b]).wait()
        plsc.subcore_barrier()

        def wb(k, _):
            pltpu.sync_copy(acc_sh.at[pl.ds(s * nslice + k * 128, 128)], rows[0])

            @pl.when(c == 0)
            def _():
                pltpu.sync_copy(
                    rows[0], out_a.at[pl.ds(s * nslice + k * 128, 128)])

            @pl.when(c == 1)
            def _():
                pltpu.sync_copy(
                    rows[0], out_b.at[pl.ds(s * nslice + k * 128, 128)])
            return 0
        lax.fori_loop(0, nslice // 128, wb, 0)

    @functools.partial(
        pl.kernel,
        out_type=(sds((n_pad,), jnp.float32), sds((n_pad,), jnp.float32)),
        mesh=mesh,
        compiler_params=sc_params,
        scratch_types=[
            pltpu.VMEM((ec, 128), jnp.int32),
            pltpu.VMEM((ec, 128), jnp.int32),
            pltpu.VMEM((ec, 128), jnp.float32),
            pltpu.VMEM((n_pad,), jnp.float32),
            pltpu.VMEM((nslice,), jnp.float32),
            pltpu.VMEM_SHARED((n_pad,), jnp.float32),
            pltpu.VMEM_SHARED((n_pad,), jnp.float32),
            pltpu.SemaphoreType.DMA,
        ],
    )
    def agg2_kernel(t_hbm, src_hbm, dst_hbm, out_a, out_b,
                    src_v, dst_v, vals_v, t_local, buf_v, t_sh, acc_sh, sem):
        c = lax.axis_index("c")
        s = lax.axis_index("s")
        gid = c * NS + s
        _fill(buf_v, 0.0, nslice)
        pltpu.sync_copy(buf_v, acc_sh.at[pl.ds(s * nslice, nslice)])
        pltpu.sync_copy(t_hbm.at[pl.ds(s * nslice, nslice)],
                        t_sh.at[pl.ds(s * nslice, nslice)])
        pltpu.sync_copy(src_hbm.at[pl.ds(gid * ec, ec)], src_v)
        pltpu.sync_copy(dst_hbm.at[pl.ds(gid * ec, ec)], dst_v)
        plsc.subcore_barrier()
        pltpu.sync_copy(t_sh, t_local)

        def gather_body(i, _):
            j = i // 8
            q = pl.multiple_of((i % 8) * 16, 16)
            idx = src_v[j, pl.ds(q, 16)]
            vals_v[j, pl.ds(q, 16)] = plsc.load_gather(t_local, [idx])
            return 0
        lax.fori_loop(0, ec * 8, gather_body, 0)

        def fire(i, _):
            pltpu.async_copy(vals_v.at[i], acc_sh.at[dst_v.at[i]], sem, add=True)
            return 0
        lax.fori_loop(0, ec, fire, 0)

        def drain(i, _):
            pltpu.make_async_copy(vals_v.at[i], acc_sh.at[dst_v.at[i]], sem).wait()
            return 0
        lax.fori_loop(0, ec, drain, 0)
        plsc.subcore_barrier()
        pltpu.sync_copy(acc_sh.at[pl.ds(s * nslice, nslice)], buf_v)

        @pl.when(c == 0)
        def _():
            pltpu.sync_copy(buf_v, out_a.at[pl.ds(s * nslice, nslice)])

        @pl.when(c == 1)
        def _():
            pltpu.sync_copy(buf_v, out_b.at[pl.ds(s * nslice, nslice)])

    out_pad = 8192
    opt = out_pad // NW

    @functools.partial(
        pl.kernel,
        out_type=sds((out_pad,), jnp.float32),
        mesh=mesh,
        compiler_params=sc_params,
        scratch_types=[
            pltpu.VMEM((nslice,), jnp.float32),
            pltpu.VMEM((nslice,), jnp.float32),
            pltpu.VMEM((nslice,), jnp.float32),
            pltpu.VMEM((nslice,), jnp.float32),
            pltpu.VMEM((16,), jnp.float32),
            pltpu.VMEM((2, 128), jnp.int32),
            pltpu.VMEM((opt,), jnp.float32),
            pltpu.VMEM_SHARED((n_pad,), jnp.float32),
            pltpu.SemaphoreType.DMA,
        ],
    )
    def fin_kernel(acca_hbm, accb_hbm, t_hbm, d_hbm, b2_hbm, out_hbm,
                   va, vb, vt, vd, vb2, idx_v, og_v, h2_sh, sem):
        c = lax.axis_index("c")
        s = lax.axis_index("s")
        base = s * nslice
        pltpu.sync_copy(acca_hbm.at[pl.ds(base, nslice)], va)
        pltpu.sync_copy(accb_hbm.at[pl.ds(base, nslice)], vb)
        pltpu.sync_copy(t_hbm.at[pl.ds(base, nslice)], vt)
        pltpu.sync_copy(d_hbm.at[pl.ds(base, nslice)], vd)
        pltpu.sync_copy(b2_hbm, vb2)

        def comp(i, _):
            q = pl.ds(pl.multiple_of(i * 16, 16), 16)
            va[q] = jnp.maximum((va[q] + vb[q] + vt[q]) * vd[q] + vb2[...],
                                0.0)
            return 0
        lax.fori_loop(0, nslice // 16, comp, 0)
        pltpu.sync_copy(va, h2_sh.at[pl.ds(base, nslice)])
        plsc.subcore_barrier()

        gid = c * NS + s
        obase = gid * opt

        def mkidx(i, _):
            io = lax.iota(jnp.int32, 16) + (obase + i * 16)
            qd = io // 12
            node = jnp.minimum(qd * 15 + 3 + (io - qd * 12), n_pad - 1)
            idx_v[i // 8, pl.ds(pl.multiple_of((i % 8) * 16, 16), 16)] = node
            return 0
        lax.fori_loop(0, opt // 16, mkidx, 0)
        for k in range(opt // 128):
            pltpu.async_copy(h2_sh.at[idx_v.at[k]],
                             og_v.at[pl.ds(k * 128, 128)], sem)
        for k in range(opt // 128):
            pltpu.make_async_copy(h2_sh.at[idx_v.at[k]],
                                  og_v.at[pl.ds(k * 128, 128)], sem).wait()
        pltpu.sync_copy(og_v, out_hbm.at[pl.ds(obase, opt)])

    return deg_kernel, agg1_kernel, agg2_kernel, fin_kernel


def kernel(obs, edge_index, W1, b1, W2, b2):
    n, fin = obs.shape
    fh = W1.shape[1]
    e = edge_index.shape[1]

    n_pad = ((n + NS * 128 - 1) // (NS * 128)) * (NS * 128)
    e_pad = ((e + NW * 1024 - 1) // (NW * 1024)) * (NW * 1024)
    ec = e_pad // NW // 128
    rows128 = n_pad // 128

    src = edge_index[0].astype(jnp.int32)
    dst = edge_index[1].astype(jnp.int32)
    src2d = jnp.concatenate(
        [src, jnp.zeros((e_pad - e,), jnp.int32)]).reshape(e_pad // 128, 128)
    dst2d = jnp.concatenate(
        [dst, jnp.full((e_pad - e,), n, jnp.int32)]).reshape(e_pad // 128, 128)
    obs_p = jnp.pad(obs, ((0, n_pad - n), (0, 0)))

    deg_kernel, agg1_kernel, agg2_kernel, fin_kernel = _make_sc_kernels(
        n_pad, ec, fh)

    dega, degb = deg_kernel(dst2d)

    def _mm(obs_ref, w1_ref, xw_ref):
        xw_ref[...] = jnp.dot(obs_ref[...].astype(jnp.bfloat16),
                              w1_ref[...].astype(jnp.bfloat16),
                              preferred_element_type=jnp.float32)

    xw = pl.pallas_call(
        _mm,
        out_shape=jax.ShapeDtypeStruct((n_pad, fh), jnp.float32),
    )(obs_p, W1)

    def _scale(xw_ref, dega_ref, degb_ref, u_ref, d_ref):
        deg = dega_ref[...] + degb_ref[...] + 1.0
        d = lax.rsqrt(deg)
        u_ref[...] = jnp.reshape(xw_ref[...], (rows128, 128, fh)) * d[:, :, None]
        d_ref[...] = d

    u3, dmat = pl.pallas_call(
        _scale,
        out_shape=(jax.ShapeDtypeStruct((rows128, 128, fh), jnp.float32),
                   jax.ShapeDtypeStruct((rows128, 128), jnp.float32)),
    )(xw, dega.reshape(rows128, 128), degb.reshape(rows128, 128))

    acc1a, acc1b = agg1_kernel(u3.reshape(n_pad, fh), src2d, dst2d)

    def _mid(acca_ref, accb_ref, u_ref, d_ref, b1_ref, w2_ref, t_ref):
        d = d_ref[...]
        h1 = jnp.maximum(
            (acca_ref[...] + accb_ref[...] + u_ref[...]) * d[:, :, None]
            + b1_ref[...], 0.0)
        h1b = h1.astype(jnp.bfloat16).astype(jnp.float32)
        w2b = w2_ref[...].astype(jnp.bfloat16).astype(jnp.float32)
        t_ref[...] = jnp.sum(h1b * w2b, axis=2) * d

    t = pl.pallas_call(
        _mid,
        out_shape=jax.ShapeDtypeStruct((rows128, 128), jnp.float32),
    )(acc1a.reshape(rows128, 128, fh), acc1b.reshape(rows128, 128, fh),
      u3, dmat, b1.reshape(1, 1, fh), W2.reshape(1, 1, fh))

    acc2a, acc2b = agg2_kernel(t.reshape(n_pad), src2d, dst2d)

    out = fin_kernel(acc2a, acc2b, t.reshape(n_pad), dmat.reshape(n_pad),
                     jnp.broadcast_to(b2, (16,)))
    return out[:(n // 15) * 12]


# speedup vs baseline: 1.1037x; 1.1037x over previous
import functools

import jax
import jax.numpy as jnp
from jax import lax
from jax.experimental import pallas as pl
from jax.experimental.pallas import tpu as pltpu
from jax.experimental.pallas import tpu_sc as plsc

NC = 2
NS = 16
NW = NC * NS


def _fill(ref, value, n):
    def body(i, _):
        ref[pl.ds(pl.multiple_of(i * 16, 16), 16)] = jnp.full(
            (16,), value, jnp.float32)
        return 0
    lax.fori_loop(0, n // 16, body, 0)


def _fill2d(ref, value, rows, cols):
    def body(i, _):
        r = i // (cols // 16)
        q = pl.multiple_of((i % (cols // 16)) * 16, 16)
        ref[r, pl.ds(q, 16)] = jnp.full((16,), value, jnp.float32)
        return 0
    lax.fori_loop(0, rows * (cols // 16), body, 0)


def _make_sc_kernels(n_pad, ec, fh, n_out):
    nslice = n_pad // NS
    mesh = plsc.VectorSubcoreMesh(core_axis_name="c", subcore_axis_name="s")
    sc_params = pltpu.CompilerParams(
        use_tc_tiling_on_sc=False, needs_layout_passes=False)
    sds = jax.ShapeDtypeStruct

    @functools.partial(
        pl.kernel,
        out_type=(sds((n_pad,), jnp.float32), sds((n_pad,), jnp.float32)),
        mesh=mesh,
        compiler_params=sc_params,
        scratch_types=[
            pltpu.VMEM((ec, 128), jnp.int32),
            pltpu.VMEM((128,), jnp.float32),
            pltpu.VMEM((nslice,), jnp.float32),
            pltpu.VMEM_SHARED((n_pad,), jnp.float32),
            pltpu.SemaphoreType.DMA,
        ],
    )
    def deg_kernel(dst_hbm, out_a, out_b, dst_v, ones_v, buf_v, acc_sh, sem):
        c = lax.axis_index("c")
        s = lax.axis_index("s")
        gid = c * NS + s
        _fill(ones_v, 1.0, 128)
        _fill(buf_v, 0.0, nslice)
        pltpu.sync_copy(buf_v, acc_sh.at[pl.ds(s * nslice, nslice)])
        plsc.subcore_barrier()
        pltpu.sync_copy(dst_hbm.at[pl.ds(gid * ec, ec)], dst_v)

        def fire(i, _):
            pltpu.async_copy(ones_v, acc_sh.at[dst_v.at[i]], sem, add=True)
            return 0
        lax.fori_loop(0, ec, fire, 0)

        def drain(i, _):
            pltpu.make_async_copy(ones_v, acc_sh.at[dst_v.at[i]], sem).wait()
            return 0
        lax.fori_loop(0, ec, drain, 0)
        plsc.subcore_barrier()
        pltpu.sync_copy(acc_sh.at[pl.ds(s * nslice, nslice)], buf_v)

        @pl.when(c == 0)
        def _():
            pltpu.sync_copy(buf_v, out_a.at[pl.ds(s * nslice, nslice)])

        @pl.when(c == 1)
        def _():
            pltpu.sync_copy(buf_v, out_b.at[pl.ds(s * nslice, nslice)])

    nb = 5
    assert ec % nb == 0

    @functools.partial(
        pl.kernel,
        out_type=(sds((n_pad, fh), jnp.float32), sds((n_pad, fh), jnp.float32)),
        mesh=mesh,
        compiler_params=sc_params,
        scratch_types=[
            pltpu.VMEM_SHARED((n_pad, fh), jnp.float32),
            pltpu.VMEM_SHARED((n_pad, fh), jnp.float32),
        ] + [pltpu.VMEM((128, fh), jnp.float32) for _ in range(nb)]
          + [pltpu.VMEM((1, 128), jnp.int32) for _ in range(2 * nb)]
          + [pltpu.SemaphoreType.DMA for _ in range(4 * nb)],
    )
    def agg1_kernel(u_hbm, src_hbm, dst_hbm, out_a, out_b,
                    u_sh, acc_sh, *rest):
        rows = rest[:nb]
        srcr = rest[nb:2 * nb]
        dstr = rest[2 * nb:3 * nb]
        gsem = rest[3 * nb:4 * nb]
        ssem = rest[4 * nb:5 * nb]
        srcsem = rest[5 * nb:6 * nb]
        dstsem = rest[6 * nb:7 * nb]
        c = lax.axis_index("c")
        s = lax.axis_index("s")
        gid = c * NS + s

        def src_load(b, j):
            pltpu.async_copy(src_hbm.at[pl.ds(gid * ec + j, 1)], srcr[b],
                             srcsem[b])

        def src_wait(b):
            pltpu.make_async_copy(src_hbm.at[pl.ds(gid * ec, 1)], srcr[b],
                                  srcsem[b]).wait()

        def dst_load(b, j):
            pltpu.async_copy(dst_hbm.at[pl.ds(gid * ec + j, 1)], dstr[b],
                             dstsem[b])

        def dst_wait(b):
            pltpu.make_async_copy(dst_hbm.at[pl.ds(gid * ec, 1)], dstr[b],
                                  dstsem[b]).wait()

        pltpu.sync_copy(u_hbm.at[pl.ds(s * nslice, nslice)],
                        u_sh.at[pl.ds(s * nslice, nslice)])
        _fill2d(rows[0], 0.0, 128, fh)

        def zero(k, _):
            pltpu.sync_copy(rows[0], acc_sh.at[pl.ds(s * nslice + k * 128, 128)])
            return 0
        lax.fori_loop(0, nslice // 128, zero, 0)
        plsc.subcore_barrier()

        def gather(b, j):
            pltpu.async_copy(u_sh.at[srcr[b].at[0]], rows[b], gsem[b])

        for b in range(nb):
            src_load(b, b)
            dst_load(b, b)
        for b in range(nb):
            src_wait(b)
            gather(b, b)

        def round_body(r, _):
            for b in range(nb):
                j = r * nb + b
                pltpu.make_async_copy(
                    u_sh.at[srcr[b].at[0]], rows[b], gsem[b]).wait()

                @pl.when(j + nb < ec)
                def _(b=b, j=j):
                    src_load(b, j + nb)
                dst_wait(b)
                pltpu.async_copy(
                    rows[b], acc_sh.at[dstr[b].at[0]], ssem[b], add=True)
            for b in range(nb):
                j = r * nb + b

                @pl.when(j + nb < ec)
                def _(b=b, j=j):
                    pltpu.make_async_copy(
                        rows[b], acc_sh.at[dstr[b].at[0]], ssem[b]).wait()
                    dst_load(b, j + nb)
                    src_wait(b)
                    gather(b, j + nb)
            return 0
        lax.fori_loop(0, ec // nb, round_body, 0)
        for b in range(nb):
            pltpu.make_async_copy(
                rows[b], acc_sh.at[dstr[b].at[0]], ssem[b]).wait()
        plsc.subcore_barrier()

        def wb(k, _):
            pltpu.sync_copy(acc_sh.at[pl.ds(s * nslice + k * 128, 128)], rows[0])

            @pl.when(c == 0)
            def _():
                pltpu.sync_copy(
                    rows[0], out_a.at[pl.ds(s * nslice + k * 128, 128)])

            @pl.when(c == 1)
            def _():
                pltpu.sync_copy(
                    rows[0], out_b.at[pl.ds(s * nslice + k * 128, 128)])
            return 0
        lax.fori_loop(0, nslice // 128, wb, 0)

    @functools.partial(
        pl.kernel,
        out_type=(sds((n_pad,), jnp.float32), sds((n_pad,), jnp.float32)),
        mesh=mesh,
        compiler_params=sc_params,
        scratch_types=[
            pltpu.VMEM((ec, 128), jnp.int32),
            pltpu.VMEM((ec, 128), jnp.int32),
            pltpu.VMEM((ec, 128), jnp.float32),
            pltpu.VMEM((n_pad,), jnp.float32),
            pltpu.VMEM((nslice,), jnp.float32),
            pltpu.VMEM_SHARED((n_pad,), jnp.float32),
            pltpu.VMEM_SHARED((n_pad,), jnp.float32),
            pltpu.SemaphoreType.DMA,
        ],
    )
    def agg2_kernel(t_hbm, src_hbm, dst_hbm, out_a, out_b,
                    src_v, dst_v, vals_v, t_local, buf_v, t_sh, acc_sh, sem):
        c = lax.axis_index("c")
        s = lax.axis_index("s")
        gid = c * NS + s
        _fill(buf_v, 0.0, nslice)
        pltpu.sync_copy(buf_v, acc_sh.at[pl.ds(s * nslice, nslice)])
        pltpu.sync_copy(t_hbm.at[pl.ds(s * nslice, nslice)],
                        t_sh.at[pl.ds(s * nslice, nslice)])
        pltpu.sync_copy(src_hbm.at[pl.ds(gid * ec, ec)], src_v)
        pltpu.sync_copy(dst_hbm.at[pl.ds(gid * ec, ec)], dst_v)
        plsc.subcore_barrier()
        pltpu.sync_copy(t_sh, t_local)

        def gather_body(i, _):
            j = i // 8
            q = pl.multiple_of((i % 8) * 16, 16)
            idx = src_v[j, pl.ds(q, 16)]
            vals_v[j, pl.ds(q, 16)] = plsc.load_gather(t_local, [idx])
            return 0
        lax.fori_loop(0, ec * 8, gather_body, 0)

        def fire(i, _):
            pltpu.async_copy(vals_v.at[i], acc_sh.at[dst_v.at[i]], sem, add=True)
            return 0
        lax.fori_loop(0, ec, fire, 0)

        def drain(i, _):
            pltpu.make_async_copy(vals_v.at[i], acc_sh.at[dst_v.at[i]], sem).wait()
            return 0
        lax.fori_loop(0, ec, drain, 0)
        plsc.subcore_barrier()
        pltpu.sync_copy(acc_sh.at[pl.ds(s * nslice, nslice)], buf_v)

        @pl.when(c == 0)
        def _():
            pltpu.sync_copy(buf_v, out_a.at[pl.ds(s * nslice, nslice)])

        @pl.when(c == 1)
        def _():
            pltpu.sync_copy(buf_v, out_b.at[pl.ds(s * nslice, nslice)])

    opt = 256
    last = n_out - (NW - 1) * opt
    assert 0 < last <= opt and last % 8 == 0 and n_out <= NW * opt

    @functools.partial(
        pl.kernel,
        out_type=sds((n_out,), jnp.float32),
        mesh=mesh,
        compiler_params=sc_params,
        scratch_types=[
            pltpu.VMEM((nslice,), jnp.float32),
            pltpu.VMEM((nslice,), jnp.float32),
            pltpu.VMEM((nslice,), jnp.float32),
            pltpu.VMEM((nslice,), jnp.float32),
            pltpu.VMEM((16,), jnp.float32),
            pltpu.VMEM((2, 128), jnp.int32),
            pltpu.VMEM((opt,), jnp.float32),
            pltpu.VMEM_SHARED((n_pad,), jnp.float32),
            pltpu.SemaphoreType.DMA,
        ],
    )
    def fin_kernel(acca_hbm, accb_hbm, t_hbm, d_hbm, b2_hbm, out_hbm,
                   va, vb, vt, vd, vb2, idx_v, og_v, h2_sh, sem):
        c = lax.axis_index("c")
        s = lax.axis_index("s")
        base = s * nslice
        pltpu.sync_copy(acca_hbm.at[pl.ds(base, nslice)], va)
        pltpu.sync_copy(accb_hbm.at[pl.ds(base, nslice)], vb)
        pltpu.sync_copy(t_hbm.at[pl.ds(base, nslice)], vt)
        pltpu.sync_copy(d_hbm.at[pl.ds(base, nslice)], vd)
        pltpu.sync_copy(b2_hbm, vb2)

        def comp(i, _):
            q = pl.ds(pl.multiple_of(i * 16, 16), 16)
            va[q] = jnp.maximum((va[q] + vb[q] + vt[q]) * vd[q] + vb2[...],
                                0.0)
            return 0
        lax.fori_loop(0, nslice // 16, comp, 0)
        pltpu.sync_copy(va, h2_sh.at[pl.ds(base, nslice)])
        plsc.subcore_barrier()

        gid = c * NS + s
        obase = gid * opt

        def mkidx(i, _):
            io = lax.iota(jnp.int32, 16) + (obase + i * 16)
            qd = io // 12
            node = jnp.minimum(qd * 15 + 3 + (io - qd * 12), n_pad - 1)
            idx_v[i // 8, pl.ds(pl.multiple_of((i % 8) * 16, 16), 16)] = node
            return 0
        lax.fori_loop(0, opt // 16, mkidx, 0)
        for k in range(opt // 128):
            pltpu.async_copy(h2_sh.at[idx_v.at[k]],
                             og_v.at[pl.ds(k * 128, 128)], sem)
        for k in range(opt // 128):
            pltpu.make_async_copy(h2_sh.at[idx_v.at[k]],
                                  og_v.at[pl.ds(k * 128, 128)], sem).wait()

        @pl.when(gid < NW - 1)
        def _():
            pltpu.sync_copy(og_v, out_hbm.at[pl.ds(obase, opt)])

        @pl.when(gid == NW - 1)
        def _():
            pltpu.sync_copy(og_v.at[pl.ds(0, last)],
                            out_hbm.at[pl.ds(obase, last)])

    return deg_kernel, agg1_kernel, agg2_kernel, fin_kernel


def kernel(obs, edge_index, W1, b1, W2, b2):
    n, fin = obs.shape
    fh = W1.shape[1]
    e = edge_index.shape[1]

    n_pad = ((n + NS * 128 - 1) // (NS * 128)) * (NS * 128)
    e_pad = ((e + NW * 1024 - 1) // (NW * 1024)) * (NW * 1024)
    ec = e_pad // NW // 128
    rows128 = n_pad // 128

    src = edge_index[0].astype(jnp.int32)
    dst = edge_index[1].astype(jnp.int32)
    src2d = jnp.concatenate(
        [src, jnp.zeros((e_pad - e,), jnp.int32)]).reshape(e_pad // 128, 128)
    dst2d = jnp.concatenate(
        [dst, jnp.full((e_pad - e,), n, jnp.int32)]).reshape(e_pad // 128, 128)
    obs_p = jnp.pad(obs, ((0, n_pad - n), (0, 0)))

    n_out = (n // 15) * 12
    deg_kernel, agg1_kernel, agg2_kernel, fin_kernel = _make_sc_kernels(
        n_pad, ec, fh, n_out)

    dega, degb = deg_kernel(dst2d)

    def _mm(obs_ref, w1_ref, xw_ref):
        xw_ref[...] = jnp.dot(obs_ref[...].astype(jnp.bfloat16),
                              w1_ref[...].astype(jnp.bfloat16),
                              preferred_element_type=jnp.float32)

    xw = pl.pallas_call(
        _mm,
        out_shape=jax.ShapeDtypeStruct((n_pad, fh), jnp.float32),
    )(obs_p, W1)

    def _scale(xw_ref, dega_ref, degb_ref, u_ref, d_ref):
        deg = dega_ref[...] + degb_ref[...] + 1.0
        d = lax.rsqrt(deg)
        u3 = jnp.reshape(xw_ref[...], (rows128, 128, fh)) * d[:, :, None]
        u_ref[...] = jnp.reshape(u3, (n_pad, fh))
        d_ref[...] = d

    u, dmat = pl.pallas_call(
        _scale,
        out_shape=(jax.ShapeDtypeStruct((n_pad, fh), jnp.float32),
                   jax.ShapeDtypeStruct((rows128, 128), jnp.float32)),
    )(xw, dega.reshape(rows128, 128), degb.reshape(rows128, 128))

    acc1a, acc1b = agg1_kernel(u, src2d, dst2d)

    def _mid(acca_ref, accb_ref, u_ref, d_ref, b1_ref, w2_ref, t_ref):
        d = d_ref[...]
        acc3 = jnp.reshape(
            acca_ref[...] + accb_ref[...] + u_ref[...], (rows128, 128, fh))
        h1 = jnp.maximum(acc3 * d[:, :, None] + b1_ref[...], 0.0)
        h1b = h1.astype(jnp.bfloat16).astype(jnp.float32)
        w2b = w2_ref[...].astype(jnp.bfloat16).astype(jnp.float32)
        t_ref[...] = jnp.sum(h1b * w2b, axis=2) * d

    t = pl.pallas_call(
        _mid,
        out_shape=jax.ShapeDtypeStruct((rows128, 128), jnp.float32),
    )(acc1a, acc1b, u, dmat, b1.reshape(1, 1, fh), W2.reshape(1, 1, fh))

    acc2a, acc2b = agg2_kernel(t.reshape(n_pad), src2d, dst2d)

    return fin_kernel(acc2a, acc2b, t.reshape(n_pad), dmat.reshape(n_pad),
                      jnp.broadcast_to(b2, (16,)))


# speedup vs baseline: 1.1603x; 1.0512x over previous
import functools

import jax
import jax.numpy as jnp
from jax import lax
from jax.experimental import pallas as pl
from jax.experimental.pallas import tpu as pltpu
from jax.experimental.pallas import tpu_sc as plsc

NC = 2
NS = 16
NW = NC * NS


def _fill(ref, value, n):
    def body(i, _):
        ref[pl.ds(pl.multiple_of(i * 16, 16), 16)] = jnp.full(
            (16,), value, jnp.float32)
        return 0
    lax.fori_loop(0, n // 16, body, 0)


def _fill2d(ref, value, rows, cols):
    def body(i, _):
        r = i // (cols // 16)
        q = pl.multiple_of((i % (cols // 16)) * 16, 16)
        ref[r, pl.ds(q, 16)] = jnp.full((16,), value, jnp.float32)
        return 0
    lax.fori_loop(0, rows * (cols // 16), body, 0)


def _make_sc_kernels(n_pad, ec, fh, n_out):
    nslice = n_pad // NS
    ect = NW * ec
    mesh = plsc.VectorSubcoreMesh(core_axis_name="c", subcore_axis_name="s")
    sc_params = pltpu.CompilerParams(
        use_tc_tiling_on_sc=False, needs_layout_passes=False)
    sds = jax.ShapeDtypeStruct

    @functools.partial(
        pl.kernel,
        out_type=(sds((n_pad,), jnp.float32), sds((n_pad,), jnp.float32)),
        mesh=mesh,
        compiler_params=sc_params,
        scratch_types=[
            pltpu.VMEM((ec, 128), jnp.int32),
            pltpu.VMEM((128,), jnp.float32),
            pltpu.VMEM((nslice,), jnp.float32),
            pltpu.VMEM_SHARED((n_pad,), jnp.float32),
            pltpu.SemaphoreType.DMA,
        ],
    )
    def deg_kernel(ei_hbm, out_a, out_b, dst_v, ones_v, buf_v, acc_sh, sem):
        c = lax.axis_index("c")
        s = lax.axis_index("s")
        gid = c * NS + s
        _fill(ones_v, 1.0, 128)
        _fill(buf_v, 0.0, nslice)
        pltpu.sync_copy(buf_v, acc_sh.at[pl.ds(s * nslice, nslice)])
        plsc.subcore_barrier()
        pltpu.sync_copy(ei_hbm.at[pl.ds(ect + gid * ec, ec)], dst_v)

        def fire(i, _):
            pltpu.async_copy(ones_v, acc_sh.at[dst_v.at[i]], sem, add=True)
            return 0
        lax.fori_loop(0, ec, fire, 0)

        def drain(i, _):
            pltpu.make_async_copy(ones_v, acc_sh.at[dst_v.at[i]], sem).wait()
            return 0
        lax.fori_loop(0, ec, drain, 0)
        plsc.subcore_barrier()
        pltpu.sync_copy(acc_sh.at[pl.ds(s * nslice, nslice)], buf_v)

        @pl.when(c == 0)
        def _():
            pltpu.sync_copy(buf_v, out_a.at[pl.ds(s * nslice, nslice)])

        @pl.when(c == 1)
        def _():
            pltpu.sync_copy(buf_v, out_b.at[pl.ds(s * nslice, nslice)])

    nb = 5
    assert ec % nb == 0

    @functools.partial(
        pl.kernel,
        out_type=(sds((n_pad, fh), jnp.float32), sds((n_pad, fh), jnp.float32)),
        mesh=mesh,
        compiler_params=sc_params,
        scratch_types=[
            pltpu.VMEM_SHARED((n_pad, fh), jnp.float32),
            pltpu.VMEM_SHARED((n_pad, fh), jnp.float32),
        ] + [pltpu.VMEM((128, fh), jnp.float32) for _ in range(nb)]
          + [pltpu.VMEM((1, 128), jnp.int32) for _ in range(2 * nb)]
          + [pltpu.SemaphoreType.DMA for _ in range(4 * nb)],
    )
    def agg1_kernel(u_hbm, ei_hbm, out_a, out_b,
                    u_sh, acc_sh, *rest):
        rows = rest[:nb]
        srcr = rest[nb:2 * nb]
        dstr = rest[2 * nb:3 * nb]
        gsem = rest[3 * nb:4 * nb]
        ssem = rest[4 * nb:5 * nb]
        srcsem = rest[5 * nb:6 * nb]
        dstsem = rest[6 * nb:7 * nb]
        c = lax.axis_index("c")
        s = lax.axis_index("s")
        gid = c * NS + s

        def src_load(b, j):
            pltpu.async_copy(ei_hbm.at[pl.ds(gid * ec + j, 1)], srcr[b],
                             srcsem[b])

        def src_wait(b):
            pltpu.make_async_copy(ei_hbm.at[pl.ds(gid * ec, 1)], srcr[b],
                                  srcsem[b]).wait()

        def dst_load(b, j):
            pltpu.async_copy(ei_hbm.at[pl.ds(ect + gid * ec + j, 1)], dstr[b],
                             dstsem[b])

        def dst_wait(b):
            pltpu.make_async_copy(ei_hbm.at[pl.ds(ect, 1)], dstr[b],
                                  dstsem[b]).wait()

        pltpu.sync_copy(u_hbm.at[pl.ds(s * nslice, nslice)],
                        u_sh.at[pl.ds(s * nslice, nslice)])
        _fill2d(rows[0], 0.0, 128, fh)

        def zero(k, _):
            pltpu.sync_copy(rows[0], acc_sh.at[pl.ds(s * nslice + k * 128, 128)])
            return 0
        lax.fori_loop(0, nslice // 128, zero, 0)
        plsc.subcore_barrier()

        def gather(b, j):
            pltpu.async_copy(u_sh.at[srcr[b].at[0]], rows[b], gsem[b])

        for b in range(nb):
            src_load(b, b)
            dst_load(b, b)
        for b in range(nb):
            src_wait(b)
            gather(b, b)

        def round_body(r, _):
            for b in range(nb):
                j = r * nb + b
                pltpu.make_async_copy(
                    u_sh.at[srcr[b].at[0]], rows[b], gsem[b]).wait()

                @pl.when(j + nb < ec)
                def _(b=b, j=j):
                    src_load(b, j + nb)
                dst_wait(b)
                pltpu.async_copy(
                    rows[b], acc_sh.at[dstr[b].at[0]], ssem[b], add=True)
            for b in range(nb):
                j = r * nb + b

                @pl.when(j + nb < ec)
                def _(b=b, j=j):
                    pltpu.make_async_copy(
                        rows[b], acc_sh.at[dstr[b].at[0]], ssem[b]).wait()
                    dst_load(b, j + nb)
                    src_wait(b)
                    gather(b, j + nb)
            return 0
        lax.fori_loop(0, ec // nb, round_body, 0)
        for b in range(nb):
            pltpu.make_async_copy(
                rows[b], acc_sh.at[dstr[b].at[0]], ssem[b]).wait()
        plsc.subcore_barrier()

        def wb(k, _):
            pltpu.sync_copy(acc_sh.at[pl.ds(s * nslice + k * 128, 128)], rows[0])

            @pl.when(c == 0)
            def _():
                pltpu.sync_copy(
                    rows[0], out_a.at[pl.ds(s * nslice + k * 128, 128)])

            @pl.when(c == 1)
            def _():
                pltpu.sync_copy(
                    rows[0], out_b.at[pl.ds(s * nslice + k * 128, 128)])
            return 0
        lax.fori_loop(0, nslice // 128, wb, 0)

    @functools.partial(
        pl.kernel,
        out_type=(sds((n_pad,), jnp.float32), sds((n_pad,), jnp.float32)),
        mesh=mesh,
        compiler_params=sc_params,
        scratch_types=[
            pltpu.VMEM((ec, 128), jnp.int32),
            pltpu.VMEM((ec, 128), jnp.int32),
            pltpu.VMEM((ec, 128), jnp.float32),
            pltpu.VMEM((n_pad,), jnp.float32),
            pltpu.VMEM((nslice,), jnp.float32),
            pltpu.VMEM_SHARED((n_pad,), jnp.float32),
            pltpu.VMEM_SHARED((n_pad,), jnp.float32),
            pltpu.SemaphoreType.DMA,
        ],
    )
    def agg2_kernel(t_hbm, ei_hbm, out_a, out_b,
                    src_v, dst_v, vals_v, t_local, buf_v, t_sh, acc_sh, sem):
        c = lax.axis_index("c")
        s = lax.axis_index("s")
        gid = c * NS + s
        _fill(buf_v, 0.0, nslice)
        pltpu.sync_copy(buf_v, acc_sh.at[pl.ds(s * nslice, nslice)])
        pltpu.sync_copy(t_hbm.at[pl.ds(s * nslice, nslice)],
                        t_sh.at[pl.ds(s * nslice, nslice)])
        pltpu.sync_copy(ei_hbm.at[pl.ds(gid * ec, ec)], src_v)
        pltpu.sync_copy(ei_hbm.at[pl.ds(ect + gid * ec, ec)], dst_v)
        plsc.subcore_barrier()
        pltpu.sync_copy(t_sh, t_local)

        def gather_body(i, _):
            j = i // 8
            q = pl.multiple_of((i % 8) * 16, 16)
            idx = src_v[j, pl.ds(q, 16)]
            vals_v[j, pl.ds(q, 16)] = plsc.load_gather(t_local, [idx])
            return 0
        lax.fori_loop(0, ec * 8, gather_body, 0)

        def fire(i, _):
            pltpu.async_copy(vals_v.at[i], acc_sh.at[dst_v.at[i]], sem, add=True)
            return 0
        lax.fori_loop(0, ec, fire, 0)

        def drain(i, _):
            pltpu.make_async_copy(vals_v.at[i], acc_sh.at[dst_v.at[i]], sem).wait()
            return 0
        lax.fori_loop(0, ec, drain, 0)
        plsc.subcore_barrier()
        pltpu.sync_copy(acc_sh.at[pl.ds(s * nslice, nslice)], buf_v)

        @pl.when(c == 0)
        def _():
            pltpu.sync_copy(buf_v, out_a.at[pl.ds(s * nslice, nslice)])

        @pl.when(c == 1)
        def _():
            pltpu.sync_copy(buf_v, out_b.at[pl.ds(s * nslice, nslice)])

    opt = 256
    last = n_out - (NW - 1) * opt
    assert 0 < last <= opt and last % 8 == 0 and n_out <= NW * opt

    @functools.partial(
        pl.kernel,
        out_type=sds((n_out,), jnp.float32),
        mesh=mesh,
        compiler_params=sc_params,
        scratch_types=[
            pltpu.VMEM((nslice,), jnp.float32),
            pltpu.VMEM((nslice,), jnp.float32),
            pltpu.VMEM((nslice,), jnp.float32),
            pltpu.VMEM((nslice,), jnp.float32),
            pltpu.VMEM((16,), jnp.float32),
            pltpu.VMEM((2, 128), jnp.int32),
            pltpu.VMEM((opt,), jnp.float32),
            pltpu.VMEM_SHARED((n_pad,), jnp.float32),
            pltpu.SemaphoreType.DMA,
        ],
    )
    def fin_kernel(acca_hbm, accb_hbm, t_hbm, d_hbm, b2_hbm, out_hbm,
                   va, vb, vt, vd, vb2, idx_v, og_v, h2_sh, sem):
        c = lax.axis_index("c")
        s = lax.axis_index("s")
        base = s * nslice
        pltpu.sync_copy(acca_hbm.at[pl.ds(base, nslice)], va)
        pltpu.sync_copy(accb_hbm.at[pl.ds(base, nslice)], vb)
        pltpu.sync_copy(t_hbm.at[pl.ds(base, nslice)], vt)
        pltpu.sync_copy(d_hbm.at[pl.ds(base, nslice)], vd)
        pltpu.sync_copy(b2_hbm, vb2)

        def comp(i, _):
            q = pl.ds(pl.multiple_of(i * 16, 16), 16)
            va[q] = jnp.maximum((va[q] + vb[q] + vt[q]) * vd[q] + vb2[...],
                                0.0)
            return 0
        lax.fori_loop(0, nslice // 16, comp, 0)
        pltpu.sync_copy(va, h2_sh.at[pl.ds(base, nslice)])
        plsc.subcore_barrier()

        gid = c * NS + s
        obase = gid * opt

        def mkidx(i, _):
            io = lax.iota(jnp.int32, 16) + (obase + i * 16)
            qd = io // 12
            node = jnp.minimum(qd * 15 + 3 + (io - qd * 12), n_pad - 1)
            idx_v[i // 8, pl.ds(pl.multiple_of((i % 8) * 16, 16), 16)] = node
            return 0
        lax.fori_loop(0, opt // 16, mkidx, 0)
        for k in range(opt // 128):
            pltpu.async_copy(h2_sh.at[idx_v.at[k]],
                             og_v.at[pl.ds(k * 128, 128)], sem)
        for k in range(opt // 128):
            pltpu.make_async_copy(h2_sh.at[idx_v.at[k]],
                                  og_v.at[pl.ds(k * 128, 128)], sem).wait()

        @pl.when(gid < NW - 1)
        def _():
            pltpu.sync_copy(og_v, out_hbm.at[pl.ds(obase, opt)])

        @pl.when(gid == NW - 1)
        def _():
            pltpu.sync_copy(og_v.at[pl.ds(0, last)],
                            out_hbm.at[pl.ds(obase, last)])

    return deg_kernel, agg1_kernel, agg2_kernel, fin_kernel


def kernel(obs, edge_index, W1, b1, W2, b2):
    n, fin = obs.shape
    fh = W1.shape[1]
    e = edge_index.shape[1]

    n_pad = ((n + NS * 128 - 1) // (NS * 128)) * (NS * 128)
    e_pad = ((e + NW * 1024 - 1) // (NW * 1024)) * (NW * 1024)
    ec = e_pad // NW // 128
    rows128 = n_pad // 128

    ei2d = jnp.pad(edge_index.astype(jnp.int32), ((0, 0), (0, e_pad - e)),
                   constant_values=n).reshape(2 * (e_pad // 128), 128)
    obs_p = jnp.pad(obs, ((0, n_pad - n), (0, 0)))

    n_out = (n // 15) * 12
    deg_kernel, agg1_kernel, agg2_kernel, fin_kernel = _make_sc_kernels(
        n_pad, ec, fh, n_out)

    dega, degb = deg_kernel(ei2d)

    def _mm(obs_ref, w1_ref, xw_ref):
        xw_ref[...] = jnp.dot(obs_ref[...].astype(jnp.bfloat16),
                              w1_ref[...].astype(jnp.bfloat16),
                              preferred_element_type=jnp.float32)

    xw = pl.pallas_call(
        _mm,
        out_shape=jax.ShapeDtypeStruct((n_pad, fh), jnp.float32),
    )(obs_p, W1)

    def _scale(xw_ref, dega_ref, degb_ref, u_ref, d_ref):
        deg = dega_ref[...] + degb_ref[...] + 1.0
        d = lax.rsqrt(deg)
        u3 = jnp.reshape(xw_ref[...], (rows128, 128, fh)) * d[:, :, None]
        u_ref[...] = jnp.reshape(u3, (n_pad, fh))
        d_ref[...] = d

    u, dmat = pl.pallas_call(
        _scale,
        out_shape=(jax.ShapeDtypeStruct((n_pad, fh), jnp.float32),
                   jax.ShapeDtypeStruct((rows128, 128), jnp.float32)),
    )(xw, dega.reshape(rows128, 128), degb.reshape(rows128, 128))

    acc1a, acc1b = agg1_kernel(u, ei2d)

    def _mid(acca_ref, accb_ref, u_ref, d_ref, b1_ref, w2_ref, t_ref):
        d = d_ref[...]
        acc3 = jnp.reshape(
            acca_ref[...] + accb_ref[...] + u_ref[...], (rows128, 128, fh))
        h1 = jnp.maximum(acc3 * d[:, :, None] + b1_ref[...], 0.0)
        h1b = h1.astype(jnp.bfloat16).astype(jnp.float32)
        w2b = w2_ref[...].astype(jnp.bfloat16).astype(jnp.float32)
        t_ref[...] = jnp.sum(h1b * w2b, axis=2) * d

    t = pl.pallas_call(
        _mid,
        out_shape=jax.ShapeDtypeStruct((rows128, 128), jnp.float32),
    )(acc1a, acc1b, u, dmat, b1.reshape(1, 1, fh), W2.reshape(1, 1, fh))

    acc2a, acc2b = agg2_kernel(t.reshape(n_pad), ei2d)

    return fin_kernel(acc2a, acc2b, t.reshape(n_pad), dmat.reshape(n_pad),
                      jnp.broadcast_to(b2, (16,)))
